# Initial kernel scaffold; baseline (speedup 1.0000x reference)
import jax
import jax.numpy as jnp
from jax.experimental import pallas as pl


def kernel(x, edge_index, batch, W1l, W1r, b1, W2l, W2r, b2, Wlin, blin):
    raise NotImplementedError("write your pallas kernel here")


# speedup vs baseline: 5.0918x; 5.0918x over previous
import functools

import jax
import jax.numpy as jnp
from jax import lax
from jax.experimental import pallas as pl
from jax.experimental.pallas import tpu as pltpu
from jax.experimental.pallas import tpu_sc as plsc

N = 10000
E = 320000
D = 128
H = 128
C = 10
G = 64

NC = 2
NS = 16
NW = NC * NS
EPT = E // NW
CH = 80
NCH = EPT // CH
RA = 640
AP = NS * RA
PZ = 64
NP = RA // PZ

BLK = 1000
NBLK = N // BLK
KE = 2000
NEB = E // KE
QR = AP // 128


_MESH = plsc.VectorSubcoreMesh(
    core_axis_name="c", subcore_axis_name="s", num_cores=NC, num_subcores=NS)


@functools.partial(
    pl.kernel, mesh=_MESH,
    out_type=jax.ShapeDtypeStruct((NC * AP, D), jnp.float32),
    scratch_types=[
        pltpu.VMEM((CH,), jnp.int32),
        pltpu.VMEM((CH,), jnp.int32),
        pltpu.VMEM((CH, D), jnp.float32),
        pltpu.VMEM((PZ, D), jnp.float32),
        pltpu.VMEM_SHARED((AP, D), jnp.float32),
        pltpu.SemaphoreType.DMA,
    ])
def _agg(y_hbm, src_hbm, dst_hbm, z_hbm, out_hbm,
         idx_s, idx_d, rows, stage, acc, sem):
  c = lax.axis_index("c")
  s = lax.axis_index("s")
  wid = c * NS + s

  pltpu.sync_copy(z_hbm, stage)
  for k in range(NP):
    pltpu.sync_copy(stage, acc.at[pl.ds(s * RA + k * PZ, PZ)])
  plsc.subcore_barrier()

  base = wid * EPT

  def chunk(j, carry):
    off = base + j * CH
    pltpu.sync_copy(src_hbm.at[pl.ds(off, CH)], idx_s)
    pltpu.sync_copy(dst_hbm.at[pl.ds(off, CH)], idx_d)
    pltpu.async_copy(y_hbm.at[idx_s], rows, sem).wait()
    pltpu.sync_copy(rows, acc.at[idx_d], add=True)
    return carry

  lax.fori_loop(0, NCH, chunk, 0)
  plsc.subcore_barrier()

  for k in range(NP):
    r0 = s * RA + k * PZ
    pltpu.sync_copy(acc.at[pl.ds(r0, PZ)], stage)
    pltpu.sync_copy(stage, out_hbm.at[pl.ds(c * AP + r0, PZ)])


def _hist_body(dst_ref, out_ref, acc_ref):
  i = pl.program_id(0)

  @pl.when(i == 0)
  def _():
    acc_ref[...] = jnp.zeros_like(acc_ref)

  dd = dst_ref[0, 0, :]
  q = dd // 128
  r = dd - q * 128
  ohq = (q[:, None] == lax.broadcasted_iota(jnp.int32, (KE, QR), 1))
  ohr = (r[:, None] == lax.broadcasted_iota(jnp.int32, (KE, 128), 1))
  dn = (((0,), (0,)), ((), ()))
  acc_ref[...] += lax.dot_general(ohq.astype(jnp.float32),
                                  ohr.astype(jnp.float32), dn,
                                  preferred_element_type=jnp.float32)

  @pl.when(i == pl.num_programs(0) - 1)
  def _():
    out_ref[...] = acc_ref[...]


def _tc_hist(dst3):
  return pl.pallas_call(
      _hist_body,
      grid=(NEB,),
      in_specs=[pl.BlockSpec((1, 1, KE), lambda i: (i, 0, 0))],
      out_specs=pl.BlockSpec((QR, 128), lambda i: (0, 0)),
      out_shape=jax.ShapeDtypeStruct((QR, 128), jnp.float32),
      scratch_shapes=[pltpu.VMEM((QR, 128), jnp.float32)],
  )(dst3)


def _mm2_body(x_ref, wa_ref, wb_ref, oa_ref, ob_ref):
  xb = x_ref[...]
  oa_ref[...] = jnp.dot(xb, wa_ref[...], preferred_element_type=jnp.float32)
  ob_ref[...] = jnp.dot(xb, wb_ref[...], preferred_element_type=jnp.float32)


def _tc_a(x, w1lT, w1rT):
  return pl.pallas_call(
      _mm2_body,
      grid=(NBLK,),
      in_specs=[
          pl.BlockSpec((BLK, D), lambda i: (i, 0)),
          pl.BlockSpec((D, H), lambda i: (0, 0)),
          pl.BlockSpec((D, H), lambda i: (0, 0)),
      ],
      out_specs=[
          pl.BlockSpec((BLK, H), lambda i: (i, 0)),
          pl.BlockSpec((BLK, H), lambda i: (i, 0)),
      ],
      out_shape=[
          jax.ShapeDtypeStruct((N, H), jnp.float32),
          jax.ShapeDtypeStruct((N, H), jnp.float32),
      ],
  )(x, w1lT, w1rT)


def _combine(xl_ref, p_ref, deg_ref, b_ref):
  inv = 1.0 / jnp.maximum(deg_ref[...], 1.0)
  agg = (p_ref[0] + p_ref[1]) * inv
  return jnp.maximum(xl_ref[...] + agg + b_ref[...], 0.0)


def _b_body(xl_ref, p_ref, deg_ref, b1_ref, w2lT_ref, w2rT_ref,
            hl2_ref, y2_ref):
  h1 = _combine(xl_ref, p_ref, deg_ref, b1_ref)
  hl2_ref[...] = jnp.dot(h1, w2lT_ref[...],
                         preferred_element_type=jnp.float32)
  y2_ref[...] = jnp.dot(h1, w2rT_ref[...],
                        preferred_element_type=jnp.float32)


def _tc_b(xl, p1, degcol, b1, w2lT, w2rT):
  return pl.pallas_call(
      _b_body,
      grid=(NBLK,),
      in_specs=[
          pl.BlockSpec((BLK, H), lambda i: (i, 0)),
          pl.BlockSpec((NC, BLK, H), lambda i: (0, i, 0)),
          pl.BlockSpec((BLK, 1), lambda i: (i, 0)),
          pl.BlockSpec((1, H), lambda i: (0, 0)),
          pl.BlockSpec((H, H), lambda i: (0, 0)),
          pl.BlockSpec((H, H), lambda i: (0, 0)),
      ],
      out_specs=[
          pl.BlockSpec((BLK, H), lambda i: (i, 0)),
          pl.BlockSpec((BLK, H), lambda i: (i, 0)),
      ],
      out_shape=[
          jax.ShapeDtypeStruct((N, H), jnp.float32),
          jax.ShapeDtypeStruct((N, H), jnp.float32),
      ],
  )(xl, p1, degcol, b1, w2lT, w2rT)


def _c_body(hl2_ref, p_ref, deg_ref, b2_ref, batch_ref, wlinT_ref, blin_ref,
            out_ref, psum_ref, cnt_ref):
  i = pl.program_id(0)

  @pl.when(i == 0)
  def _():
    psum_ref[...] = jnp.zeros_like(psum_ref)
    cnt_ref[...] = jnp.zeros_like(cnt_ref)

  h2 = _combine(hl2_ref, p_ref, deg_ref, b2_ref)
  bb = batch_ref[0, 0, :]
  oh = (bb[:, None] == lax.broadcasted_iota(jnp.int32, (BLK, G), 1))
  oh = oh.astype(jnp.float32)
  dn = (((0,), (0,)), ((), ()))
  psum_ref[...] += lax.dot_general(oh, h2, dn,
                                   preferred_element_type=jnp.float32)
  cnt_ref[...] += lax.dot_general(oh, jnp.ones((BLK, H), jnp.float32), dn,
                                  preferred_element_type=jnp.float32)

  @pl.when(i == pl.num_programs(0) - 1)
  def _():
    pooled = psum_ref[...] / jnp.maximum(cnt_ref[...], 1.0)
    out_ref[...] = (jnp.dot(pooled, wlinT_ref[...],
                            preferred_element_type=jnp.float32)
                    + blin_ref[...])


def _tc_c(hl2, p2, degcol, b2, batch3, wlinT, blin):
  return pl.pallas_call(
      _c_body,
      grid=(NBLK,),
      in_specs=[
          pl.BlockSpec((BLK, H), lambda i: (i, 0)),
          pl.BlockSpec((NC, BLK, H), lambda i: (0, i, 0)),
          pl.BlockSpec((BLK, 1), lambda i: (i, 0)),
          pl.BlockSpec((1, H), lambda i: (0, 0)),
          pl.BlockSpec((1, 1, BLK), lambda i: (i, 0, 0)),
          pl.BlockSpec((H, C), lambda i: (0, 0)),
          pl.BlockSpec((1, C), lambda i: (0, 0)),
      ],
      out_specs=pl.BlockSpec((G, C), lambda i: (0, 0)),
      out_shape=jax.ShapeDtypeStruct((G, C), jnp.float32),
      scratch_shapes=[
          pltpu.VMEM((G, H), jnp.float32),
          pltpu.VMEM((G, H), jnp.float32),
      ],
  )(hl2, p2, degcol, b2, batch3, wlinT, blin)


def kernel(x, edge_index, batch, W1l, W1r, b1, W2l, W2r, b2, Wlin, blin):
  src = edge_index[0]
  dst = edge_index[1]
  z128 = jnp.zeros((PZ, D), jnp.float32)

  deg2d = _tc_hist(dst.reshape(NEB, 1, KE))
  degcol = deg2d.reshape(AP, 1)[:N]

  xl, y1 = _tc_a(x, W1l.T, W1r.T)
  p1 = _agg(y1, src, dst, z128).reshape(NC, AP, D)[:, :N, :]

  hl2, y2 = _tc_b(xl, p1, degcol, b1.reshape(1, H), W2l.T, W2r.T)
  p2 = _agg(y2, src, dst, z128).reshape(NC, AP, D)[:, :N, :]

  batch3 = batch.reshape(NBLK, 1, BLK)
  out = _tc_c(hl2, p2, degcol, b2.reshape(1, H), batch3, Wlin.T,
              blin.reshape(1, C))
  return out


# speedup vs baseline: 10.1634x; 1.9960x over previous
import functools

import jax
import jax.numpy as jnp
from jax import lax
from jax.experimental import pallas as pl
from jax.experimental.pallas import tpu as pltpu
from jax.experimental.pallas import tpu_sc as plsc

N = 10000
E = 320000
D = 128
H = 128
C = 10
G = 64

NC = 2
NS = 16
NW = NC * NS
CH = 128
NCH = 80
KB = 16
NKB = NCH // KB
EPT = NCH * CH
EPAD = NW * EPT
RA = 640
AP = NS * RA
NPZ = RA // CH

BLK = 1000
NBLK = N // BLK
KE = 2000
NEB = E // KE
QR = AP // 128


_MESH = plsc.VectorSubcoreMesh(
    core_axis_name="c", subcore_axis_name="s", num_cores=NC, num_subcores=NS)


@functools.partial(
    pl.kernel, mesh=_MESH,
    out_type=jax.ShapeDtypeStruct((NC * AP, D), jnp.float32),
    scratch_types=[
        pltpu.VMEM((KB, CH), jnp.int32),
        pltpu.VMEM((KB, CH), jnp.int32),
        pltpu.VMEM((KB, CH), jnp.int32),
        pltpu.VMEM((KB, CH), jnp.int32),
        pltpu.VMEM((CH, D), jnp.float32),
        pltpu.VMEM((CH, D), jnp.float32),
        pltpu.VMEM_SHARED((AP, D), jnp.float32),
        pltpu.SemaphoreType.DMA,
        pltpu.SemaphoreType.DMA,
        pltpu.SemaphoreType.DMA,
    ])
def _agg(y_hbm, src_hbm, dst_hbm, z_hbm, out_hbm,
         idx_s0, idx_s1, idx_d0, idx_d1, rows0, rows1, acc,
         sem0, sem1, semz):
  c = lax.axis_index("c")
  s = lax.axis_index("s")
  wid = c * NS + s
  idx_s = [idx_s0, idx_s1]
  idx_d = [idx_d0, idx_d1]
  rows = [rows0, rows1]
  sems = [sem0, sem1]

  pltpu.sync_copy(z_hbm, rows0)
  zd = [pltpu.async_copy(rows0, acc.at[pl.ds(s * RA + p * CH, CH)], semz)
        for p in range(NPZ)]
  for d in zd:
    d.wait()
  plsc.subcore_barrier()

  def load_idx(kb):
    b = kb & 1
    pltpu.sync_copy(src_hbm.at[wid, pl.ds(kb * KB, KB)], idx_s[b])
    pltpu.sync_copy(dst_hbm.at[wid, pl.ds(kb * KB, KB)], idx_d[b])

  def gather_start(g):
    kb, k = divmod(g, KB)
    pltpu.async_copy(y_hbm.at[idx_s[kb & 1].at[k]], rows[g & 1],
                     sems[g & 1])

  def gather_wait(g):
    kb, k = divmod(g, KB)
    pltpu.make_async_copy(y_hbm.at[idx_s[kb & 1].at[k]], rows[g & 1],
                          sems[g & 1]).wait()

  load_idx(0)
  gather_start(0)
  for g in range(NCH):
    kb, k = divmod(g, KB)
    if g + 1 < NCH:
      nkb, nk = divmod(g + 1, KB)
      if nk == 0:
        load_idx(nkb)
      gather_start(g + 1)
    gather_wait(g)
    pltpu.sync_copy(rows[g & 1], acc.at[idx_d[kb & 1].at[k]], add=True)

  plsc.subcore_barrier()

  outd = [None, None]
  for p in range(NPZ):
    b = p & 1
    if outd[b] is not None:
      outd[b].wait()
    r0 = s * RA + p * CH
    pltpu.sync_copy(acc.at[pl.ds(r0, CH)], rows[b])
    outd[b] = pltpu.async_copy(rows[b], out_hbm.at[pl.ds(c * AP + r0, CH)],
                               sems[b])
  for b in range(2):
    if outd[b] is not None:
      outd[b].wait()


def _hist_body(dst_ref, out_ref, acc_ref):
  i = pl.program_id(0)

  @pl.when(i == 0)
  def _():
    acc_ref[...] = jnp.zeros_like(acc_ref)

  dd = dst_ref[0, 0, :]
  q = dd // 128
  r = dd - q * 128
  ohq = (q[:, None] == lax.broadcasted_iota(jnp.int32, (KE, QR), 1))
  ohr = (r[:, None] == lax.broadcasted_iota(jnp.int32, (KE, 128), 1))
  dn = (((0,), (0,)), ((), ()))
  acc_ref[...] += lax.dot_general(ohq.astype(jnp.float32),
                                  ohr.astype(jnp.float32), dn,
                                  preferred_element_type=jnp.float32)

  @pl.when(i == pl.num_programs(0) - 1)
  def _():
    out_ref[...] = acc_ref[...]


def _tc_hist(dst3):
  return pl.pallas_call(
      _hist_body,
      grid=(NEB,),
      in_specs=[pl.BlockSpec((1, 1, KE), lambda i: (i, 0, 0))],
      out_specs=pl.BlockSpec((QR, 128), lambda i: (0, 0)),
      out_shape=jax.ShapeDtypeStruct((QR, 128), jnp.float32),
      scratch_shapes=[pltpu.VMEM((QR, 128), jnp.float32)],
  )(dst3)


def _mm2_body(x_ref, wa_ref, wb_ref, oa_ref, ob_ref):
  xb = x_ref[...]
  oa_ref[...] = jnp.dot(xb, wa_ref[...], preferred_element_type=jnp.float32)
  ob_ref[...] = jnp.dot(xb, wb_ref[...], preferred_element_type=jnp.float32)


def _tc_a(x, w1lT, w1rT):
  return pl.pallas_call(
      _mm2_body,
      grid=(NBLK,),
      in_specs=[
          pl.BlockSpec((BLK, D), lambda i: (i, 0)),
          pl.BlockSpec((D, H), lambda i: (0, 0)),
          pl.BlockSpec((D, H), lambda i: (0, 0)),
      ],
      out_specs=[
          pl.BlockSpec((BLK, H), lambda i: (i, 0)),
          pl.BlockSpec((BLK, H), lambda i: (i, 0)),
      ],
      out_shape=[
          jax.ShapeDtypeStruct((N, H), jnp.float32),
          jax.ShapeDtypeStruct((N, H), jnp.float32),
      ],
  )(x, w1lT, w1rT)


def _combine(xl_ref, p_ref, deg_ref, b_ref):
  inv = 1.0 / jnp.maximum(deg_ref[...], 1.0)
  agg = (p_ref[0] + p_ref[1]) * inv
  return jnp.maximum(xl_ref[...] + agg + b_ref[...], 0.0)


def _b_body(xl_ref, p_ref, deg_ref, b1_ref, w2lT_ref, w2rT_ref,
            hl2_ref, y2_ref):
  h1 = _combine(xl_ref, p_ref, deg_ref, b1_ref)
  hl2_ref[...] = jnp.dot(h1, w2lT_ref[...],
                         preferred_element_type=jnp.float32)
  y2_ref[...] = jnp.dot(h1, w2rT_ref[...],
                        preferred_element_type=jnp.float32)


def _tc_b(xl, p1, degcol, b1, w2lT, w2rT):
  return pl.pallas_call(
      _b_body,
      grid=(NBLK,),
      in_specs=[
          pl.BlockSpec((BLK, H), lambda i: (i, 0)),
          pl.BlockSpec((NC, BLK, H), lambda i: (0, i, 0)),
          pl.BlockSpec((BLK, 1), lambda i: (i, 0)),
          pl.BlockSpec((1, H), lambda i: (0, 0)),
          pl.BlockSpec((H, H), lambda i: (0, 0)),
          pl.BlockSpec((H, H), lambda i: (0, 0)),
      ],
      out_specs=[
          pl.BlockSpec((BLK, H), lambda i: (i, 0)),
          pl.BlockSpec((BLK, H), lambda i: (i, 0)),
      ],
      out_shape=[
          jax.ShapeDtypeStruct((N, H), jnp.float32),
          jax.ShapeDtypeStruct((N, H), jnp.float32),
      ],
  )(xl, p1, degcol, b1, w2lT, w2rT)


def _c_body(hl2_ref, p_ref, deg_ref, b2_ref, batch_ref, wlinT_ref, blin_ref,
            out_ref, psum_ref, cnt_ref):
  i = pl.program_id(0)

  @pl.when(i == 0)
  def _():
    psum_ref[...] = jnp.zeros_like(psum_ref)
    cnt_ref[...] = jnp.zeros_like(cnt_ref)

  h2 = _combine(hl2_ref, p_ref, deg_ref, b2_ref)
  bb = batch_ref[0, 0, :]
  oh = (bb[:, None] == lax.broadcasted_iota(jnp.int32, (BLK, G), 1))
  oh = oh.astype(jnp.float32)
  dn = (((0,), (0,)), ((), ()))
  psum_ref[...] += lax.dot_general(oh, h2, dn,
                                   preferred_element_type=jnp.float32)
  cnt_ref[...] += lax.dot_general(oh, jnp.ones((BLK, H), jnp.float32), dn,
                                  preferred_element_type=jnp.float32)

  @pl.when(i == pl.num_programs(0) - 1)
  def _():
    pooled = psum_ref[...] / jnp.maximum(cnt_ref[...], 1.0)
    out_ref[...] = (jnp.dot(pooled, wlinT_ref[...],
                            preferred_element_type=jnp.float32)
                    + blin_ref[...])


def _tc_c(hl2, p2, degcol, b2, batch3, wlinT, blin):
  return pl.pallas_call(
      _c_body,
      grid=(NBLK,),
      in_specs=[
          pl.BlockSpec((BLK, H), lambda i: (i, 0)),
          pl.BlockSpec((NC, BLK, H), lambda i: (0, i, 0)),
          pl.BlockSpec((BLK, 1), lambda i: (i, 0)),
          pl.BlockSpec((1, H), lambda i: (0, 0)),
          pl.BlockSpec((1, 1, BLK), lambda i: (i, 0, 0)),
          pl.BlockSpec((H, C), lambda i: (0, 0)),
          pl.BlockSpec((1, C), lambda i: (0, 0)),
      ],
      out_specs=pl.BlockSpec((G, C), lambda i: (0, 0)),
      out_shape=jax.ShapeDtypeStruct((G, C), jnp.float32),
      scratch_shapes=[
          pltpu.VMEM((G, H), jnp.float32),
          pltpu.VMEM((G, H), jnp.float32),
      ],
  )(hl2, p2, degcol, b2, batch3, wlinT, blin)


def kernel(x, edge_index, batch, W1l, W1r, b1, W2l, W2r, b2, Wlin, blin):
  src = edge_index[0]
  dst = edge_index[1]
  z128 = jnp.zeros((CH, D), jnp.float32)

  npad = EPAD - E
  pad_i = jnp.arange(npad, dtype=jnp.int32)
  src_p = jnp.concatenate([src, pad_i % N]).reshape(NW, NCH, CH)
  dst_p = jnp.concatenate([dst, N + pad_i % (AP - N)]).reshape(NW, NCH, CH)

  deg2d = _tc_hist(dst.reshape(NEB, 1, KE))
  degcol = deg2d.reshape(AP, 1)[:N]

  xl, y1 = _tc_a(x, W1l.T, W1r.T)
  p1 = _agg(y1, src_p, dst_p, z128).reshape(NC, AP, D)[:, :N, :]

  hl2, y2 = _tc_b(xl, p1, degcol, b1.reshape(1, H), W2l.T, W2r.T)
  p2 = _agg(y2, src_p, dst_p, z128).reshape(NC, AP, D)[:, :N, :]

  batch3 = batch.reshape(NBLK, 1, BLK)
  out = _tc_c(hl2, p2, degcol, b2.reshape(1, H), batch3, Wlin.T,
              blin.reshape(1, C))
  return out


# speedup vs baseline: 11.9863x; 1.1794x over previous
import functools

import jax
import jax.numpy as jnp
from jax import lax
from jax.experimental import pallas as pl
from jax.experimental.pallas import tpu as pltpu
from jax.experimental.pallas import tpu_sc as plsc

N = 10000
E = 320000
D = 128
H = 128
C = 10
G = 64

NC = 2
NS = 16
NW = NC * NS
CH = 64
NCH = 160
KB = 32
NKB = NCH // KB
NBUF = 4
EPT = NCH * CH
EPAD = NW * EPT
RA = 640
AP = NS * RA
NPZ = RA // CH

BLK = 1000
NBLK = N // BLK
KE = 20000
NEB = E // KE
QR = AP // 128


_MESH = plsc.VectorSubcoreMesh(
    core_axis_name="c", subcore_axis_name="s", num_cores=NC, num_subcores=NS)


@functools.partial(
    pl.kernel, mesh=_MESH,
    out_type=jax.ShapeDtypeStruct((NC * AP, D), jnp.float32),
    scratch_types=(
        [pltpu.VMEM((KB, CH), jnp.int32) for _ in range(2)] +
        [pltpu.VMEM((KB, CH), jnp.int32) for _ in range(2)] +
        [pltpu.VMEM((CH, D), jnp.float32) for _ in range(NBUF)] +
        [pltpu.VMEM_SHARED((AP, D), jnp.float32)] +
        [pltpu.SemaphoreType.DMA for _ in range(2 * NBUF + 1)]))
def _agg(y_hbm, src_hbm, dst_hbm, z_hbm, out_hbm, *scr):
  idx_s = list(scr[0:2])
  idx_d = list(scr[2:4])
  rows = list(scr[4:4 + NBUF])
  acc = scr[4 + NBUF]
  sems = list(scr[5 + NBUF:5 + 2 * NBUF])
  smcs = list(scr[5 + 2 * NBUF:5 + 3 * NBUF])
  semz = scr[5 + 3 * NBUF]
  c = lax.axis_index("c")
  s = lax.axis_index("s")
  wid = c * NS + s

  pltpu.sync_copy(z_hbm, rows[0])
  zd = [pltpu.async_copy(rows[0], acc.at[pl.ds(s * RA + p * CH, CH)], semz)
        for p in range(NPZ)]
  for d in zd:
    d.wait()
  plsc.subcore_barrier()

  def load_idx(kb):
    b = kb & 1
    pltpu.sync_copy(src_hbm.at[wid, pl.ds(kb * KB, KB)], idx_s[b])
    pltpu.sync_copy(dst_hbm.at[wid, pl.ds(kb * KB, KB)], idx_d[b])

  def gather_start(g):
    kb, k = divmod(g, KB)
    pltpu.async_copy(y_hbm.at[idx_s[kb & 1].at[k]], rows[g % NBUF],
                     sems[g % NBUF])

  def gather_wait(g):
    kb, k = divmod(g, KB)
    pltpu.make_async_copy(y_hbm.at[idx_s[kb & 1].at[k]], rows[g % NBUF],
                          sems[g % NBUF]).wait()

  scat = [None] * NBUF
  load_idx(0)
  for g in range(NBUF - 1):
    gather_start(g)
  for g in range(NCH):
    b = g % NBUF
    kb, k = divmod(g, KB)
    ahead = g + NBUF - 1
    if ahead < NCH:
      akb, ak = divmod(ahead, KB)
      if ak == 0:
        load_idx(akb)
      ab = ahead % NBUF
      if scat[ab] is not None:
        scat[ab].wait()
        scat[ab] = None
      gather_start(ahead)
    gather_wait(g)
    scat[b] = pltpu.async_copy(rows[b], acc.at[idx_d[kb & 1].at[k]],
                               smcs[b], add=True)
  for b in range(NBUF):
    if scat[b] is not None:
      scat[b].wait()

  plsc.subcore_barrier()

  outd = [None] * NBUF
  for p in range(NPZ):
    b = p % NBUF
    if outd[b] is not None:
      outd[b].wait()
    r0 = s * RA + p * CH
    pltpu.sync_copy(acc.at[pl.ds(r0, CH)], rows[b])
    outd[b] = pltpu.async_copy(rows[b], out_hbm.at[pl.ds(c * AP + r0, CH)],
                               sems[b])
  for b in range(NBUF):
    if outd[b] is not None:
      outd[b].wait()


def _hist_body(dst_ref, out_ref, acc_ref):
  i = pl.program_id(0)

  @pl.when(i == 0)
  def _():
    acc_ref[...] = jnp.zeros_like(acc_ref)

  dd = dst_ref[0, 0, :]
  q = dd // 128
  r = dd - q * 128
  ohq = (q[:, None] == lax.broadcasted_iota(jnp.int32, (KE, QR), 1))
  ohr = (r[:, None] == lax.broadcasted_iota(jnp.int32, (KE, 128), 1))
  dn = (((0,), (0,)), ((), ()))
  acc_ref[...] += lax.dot_general(ohq.astype(jnp.int8),
                                  ohr.astype(jnp.int8), dn,
                                  preferred_element_type=jnp.int32
                                  ).astype(jnp.float32)

  @pl.when(i == pl.num_programs(0) - 1)
  def _():
    out_ref[...] = acc_ref[...]


def _tc_hist(dst3):
  return pl.pallas_call(
      _hist_body,
      grid=(NEB,),
      in_specs=[pl.BlockSpec((1, 1, KE), lambda i: (i, 0, 0))],
      out_specs=pl.BlockSpec((QR, 128), lambda i: (0, 0)),
      out_shape=jax.ShapeDtypeStruct((QR, 128), jnp.float32),
      scratch_shapes=[pltpu.VMEM((QR, 128), jnp.float32)],
  )(dst3)


def _mm2_body(x_ref, wa_ref, wb_ref, oa_ref, ob_ref):
  xb = x_ref[...]
  oa_ref[...] = jnp.dot(xb, wa_ref[...], preferred_element_type=jnp.float32)
  ob_ref[...] = jnp.dot(xb, wb_ref[...], preferred_element_type=jnp.float32)


def _tc_a(x, w1lT, w1rT):
  return pl.pallas_call(
      _mm2_body,
      grid=(NBLK,),
      in_specs=[
          pl.BlockSpec((BLK, D), lambda i: (i, 0)),
          pl.BlockSpec((D, H), lambda i: (0, 0)),
          pl.BlockSpec((D, H), lambda i: (0, 0)),
      ],
      out_specs=[
          pl.BlockSpec((BLK, H), lambda i: (i, 0)),
          pl.BlockSpec((BLK, H), lambda i: (i, 0)),
      ],
      out_shape=[
          jax.ShapeDtypeStruct((N, H), jnp.float32),
          jax.ShapeDtypeStruct((N, H), jnp.float32),
      ],
  )(x, w1lT, w1rT)


def _combine(xl_ref, p_ref, deg_ref, b_ref):
  inv = 1.0 / jnp.maximum(deg_ref[...], 1.0)
  agg = (p_ref[0] + p_ref[1]) * inv
  return jnp.maximum(xl_ref[...] + agg + b_ref[...], 0.0)


def _b_body(xl_ref, p_ref, deg_ref, b1_ref, w2lT_ref, w2rT_ref,
            hl2_ref, y2_ref):
  h1 = _combine(xl_ref, p_ref, deg_ref, b1_ref)
  hl2_ref[...] = jnp.dot(h1, w2lT_ref[...],
                         preferred_element_type=jnp.float32)
  y2_ref[...] = jnp.dot(h1, w2rT_ref[...],
                        preferred_element_type=jnp.float32)


def _tc_b(xl, p1, degcol, b1, w2lT, w2rT):
  return pl.pallas_call(
      _b_body,
      grid=(NBLK,),
      in_specs=[
          pl.BlockSpec((BLK, H), lambda i: (i, 0)),
          pl.BlockSpec((NC, BLK, H), lambda i: (0, i, 0)),
          pl.BlockSpec((BLK, 1), lambda i: (i, 0)),
          pl.BlockSpec((1, H), lambda i: (0, 0)),
          pl.BlockSpec((H, H), lambda i: (0, 0)),
          pl.BlockSpec((H, H), lambda i: (0, 0)),
      ],
      out_specs=[
          pl.BlockSpec((BLK, H), lambda i: (i, 0)),
          pl.BlockSpec((BLK, H), lambda i: (i, 0)),
      ],
      out_shape=[
          jax.ShapeDtypeStruct((N, H), jnp.float32),
          jax.ShapeDtypeStruct((N, H), jnp.float32),
      ],
  )(xl, p1, degcol, b1, w2lT, w2rT)


def _c_body(hl2_ref, p_ref, deg_ref, b2_ref, batch_ref, wlinT_ref, blin_ref,
            out_ref, psum_ref, cnt_ref):
  i = pl.program_id(0)

  @pl.when(i == 0)
  def _():
    psum_ref[...] = jnp.zeros_like(psum_ref)
    cnt_ref[...] = jnp.zeros_like(cnt_ref)

  h2 = _combine(hl2_ref, p_ref, deg_ref, b2_ref)
  bb = batch_ref[0, 0, :]
  oh = (bb[:, None] == lax.broadcasted_iota(jnp.int32, (BLK, G), 1))
  oh = oh.astype(jnp.float32)
  dn = (((0,), (0,)), ((), ()))
  psum_ref[...] += lax.dot_general(oh, h2, dn,
                                   preferred_element_type=jnp.float32)
  ohb = oh.astype(jnp.bfloat16)
  cnt_ref[...] += lax.dot_general(ohb, jnp.ones((BLK, H), jnp.bfloat16), dn,
                                  preferred_element_type=jnp.float32)

  @pl.when(i == pl.num_programs(0) - 1)
  def _():
    pooled = psum_ref[...] / jnp.maximum(cnt_ref[...], 1.0)
    out_ref[...] = (jnp.dot(pooled, wlinT_ref[...],
                            preferred_element_type=jnp.float32)
                    + blin_ref[...])


def _tc_c(hl2, p2, degcol, b2, batch3, wlinT, blin):
  return pl.pallas_call(
      _c_body,
      grid=(NBLK,),
      in_specs=[
          pl.BlockSpec((BLK, H), lambda i: (i, 0)),
          pl.BlockSpec((NC, BLK, H), lambda i: (0, i, 0)),
          pl.BlockSpec((BLK, 1), lambda i: (i, 0)),
          pl.BlockSpec((1, H), lambda i: (0, 0)),
          pl.BlockSpec((1, 1, BLK), lambda i: (i, 0, 0)),
          pl.BlockSpec((H, C), lambda i: (0, 0)),
          pl.BlockSpec((1, C), lambda i: (0, 0)),
      ],
      out_specs=pl.BlockSpec((G, C), lambda i: (0, 0)),
      out_shape=jax.ShapeDtypeStruct((G, C), jnp.float32),
      scratch_shapes=[
          pltpu.VMEM((G, H), jnp.float32),
          pltpu.VMEM((G, H), jnp.float32),
      ],
  )(hl2, p2, degcol, b2, batch3, wlinT, blin)


def kernel(x, edge_index, batch, W1l, W1r, b1, W2l, W2r, b2, Wlin, blin):
  src = edge_index[0]
  dst = edge_index[1]
  z128 = jnp.zeros((CH, D), jnp.float32)

  npad = EPAD - E
  pad_i = jnp.arange(npad, dtype=jnp.int32)
  src_p = jnp.concatenate([src, pad_i % N]).reshape(NW, NCH, CH)
  dst_p = jnp.concatenate([dst, N + pad_i % (AP - N)]).reshape(NW, NCH, CH)

  deg2d = _tc_hist(dst.reshape(NEB, 1, KE))
  degcol = deg2d.reshape(AP, 1)[:N]

  xl, y1 = _tc_a(x, W1l.T, W1r.T)
  p1 = _agg(y1, src_p, dst_p, z128).reshape(NC, AP, D)

  hl2, y2 = _tc_b(xl, p1, degcol, b1.reshape(1, H), W2l.T, W2r.T)
  p2 = _agg(y2, src_p, dst_p, z128).reshape(NC, AP, D)

  batch3 = batch.reshape(NBLK, 1, BLK)
  out = _tc_c(hl2, p2, degcol, b2.reshape(1, H), batch3, Wlin.T,
              blin.reshape(1, C))
  return out
